# baseline (speedup 1.0000x reference)
import functools

import jax
import jax.numpy as jnp
from jax import lax
from jax.experimental import pallas as pl
from jax.experimental.pallas import tpu as pltpu
from jax.experimental.pallas import tpu_sc as plsc

_DELTA_A = 0.1
_DELTA_R = 1.0
_ALPHA = 1.0
_BETA = 1.0
_GAMMA = 0.001


def _sc_counts(tgt_flat, lab_idx_rep, b, k, hw):
    bk = b * k
    info = plsc.get_sparse_core_info()
    nw = info.num_cores * info.num_subcores
    pairs = bk // nw
    nvec = hw // 16

    @functools.partial(
        pl.kernel,
        mesh=plsc.VectorSubcoreMesh(core_axis_name="c", subcore_axis_name="s"),
        out_type=jax.ShapeDtypeStruct((bk, 16), jnp.int32),
        scratch_types=[
            pltpu.VMEM((hw,), jnp.int32),
            pltpu.VMEM((pairs, 16), jnp.int32),
            pltpu.VMEM((pairs, 16), jnp.int32),
            pltpu.VMEM((pairs, 16), jnp.int32),
            pltpu.SemaphoreType.DMA,
        ],
    )
    def count_kernel(tgt_hbm, lidx_hbm, cnt_out, img_v, labs_v, lidx_v,
                     acc_v, sem):
        wid = lax.axis_index("s") * info.num_cores + lax.axis_index("c")
        img = (wid * pairs) // k
        pltpu.sync_copy(lidx_hbm.at[pl.ds(wid * pairs, pairs)], lidx_v)
        for q in range(pairs):
            pltpu.async_copy(tgt_hbm.at[lidx_v.at[q]], labs_v.at[q], sem).wait()
        pltpu.sync_copy(tgt_hbm.at[pl.ds(img * hw, hw)], img_v)
        splats = [labs_v[q, :] for q in range(pairs)]

        def step(it, accs):
            tv = img_v[pl.ds(it * 16, 16)]
            return tuple(
                acc + jnp.where(tv == splats[q], jnp.int32(1), jnp.int32(0))
                for q, acc in enumerate(accs)
            )

        accs = lax.fori_loop(0, nvec, step,
                             tuple(jnp.zeros((16,), jnp.int32)
                                   for _ in range(pairs)))
        for q in range(pairs):
            acc_v[q, :] = accs[q]
        pltpu.sync_copy(acc_v, cnt_out.at[pl.ds(wid * pairs, pairs)])

    return count_kernel(tgt_flat, lab_idx_rep)


def _make_tc_gather_body(k):
    def body(blk_ref, off_ref, *refs):
        o_refs = refs[:k]
        t_refs = refs[k:2 * k]
        c_ref, cn2_ref, lab_ref = refs[2 * k:]
        i = pl.program_id(0)
        lanes = lax.broadcasted_iota(jnp.int32, (1, 128), 1)
        lanes_k = lax.broadcasted_iota(jnp.int32, (1, k), 1)
        cn2row = jnp.zeros((1, k), jnp.float32)
        labrow = jnp.zeros((1, k), jnp.int32)
        for j in range(k):
            po = off_ref[i * k + j]
            mskf = (lanes == po).astype(jnp.float32)
            col = jnp.sum(o_refs[j][0] * mskf, axis=1, keepdims=True)
            c_ref[0, :, j:j + 1] = col
            cn2row += jnp.sum(col * col) * (lanes_k == j).astype(jnp.float32)
            labv = jnp.sum(t_refs[j][0] * (lanes == po).astype(jnp.int32),
                           axis=1, keepdims=True)
            labrow += labv * (lanes_k == j).astype(jnp.int32)
        cn2_ref[0] = cn2row
        lab_ref[0] = labrow
    return body


def _tc_gather(out_r, tgt_r, blk, off):
    b, d, hw = out_r.shape
    bk = blk.shape[0]
    k = bk // b

    def mk_in(j):
        return pl.BlockSpec((1, d, 128),
                            lambda i, blk, off, j=j: (i, 0, blk[i * k + j]))

    def mk_tin(j):
        return pl.BlockSpec((1, 1, 128),
                            lambda i, blk, off, j=j: (i, 0, blk[i * k + j]))

    grid_spec = pltpu.PrefetchScalarGridSpec(
        num_scalar_prefetch=2,
        grid=(b,),
        in_specs=([mk_in(j) for j in range(k)]
                  + [mk_tin(j) for j in range(k)]),
        out_specs=[
            pl.BlockSpec((1, d, k), lambda i, blk, off: (i, 0, 0)),
            pl.BlockSpec((1, 1, k), lambda i, blk, off: (i, 0, 0)),
            pl.BlockSpec((1, 1, k), lambda i, blk, off: (i, 0, 0)),
        ],
    )
    c2, cn2, lab = pl.pallas_call(
        _make_tc_gather_body(k),
        grid_spec=grid_spec,
        out_shape=[
            jax.ShapeDtypeStruct((b, d, k), jnp.float32),
            jax.ShapeDtypeStruct((b, 1, k), jnp.float32),
            jax.ShapeDtypeStruct((b, 1, k), jnp.int32),
        ],
    )(blk, off, *([out_r] * k), *([tgt_r] * k))
    return c2, cn2, lab


def _tc_body(out_ref, tgt_ref, c_ref, cn2_ref, lab_ref, res_ref, attr_ref):
    j = pl.program_id(1)
    nb = pl.num_programs(1)
    o = out_ref[0]
    t = tgt_ref[0]
    c2 = c_ref[0]
    cn2 = cn2_ref[0]
    lab = lab_ref[0]
    k_centers = c2.shape[1]

    @pl.when(j == 0)
    def _():
        attr_ref[...] = jnp.zeros_like(attr_ref)

    g = lax.dot_general(c2, o, (((0,), (0,)), ((), ())),
                        preferred_element_type=jnp.float32,
                        precision=lax.Precision.DEFAULT)
    pn2 = jnp.sum(o * o, axis=0, keepdims=True)
    sq = jnp.maximum(cn2 + pn2 - 2.0 * g, 0.0)
    hinged = jnp.maximum(jnp.sqrt(sq) - _DELTA_A, 0.0)
    hm = jnp.where(t == lab, hinged, 0.0)
    attr_ref[:, :1] += jnp.sum(hm, axis=1, keepdims=True)

    @pl.when(j == nb - 1)
    def _():
        r_i = jnp.float32(0.0)
        for jj in range(k_centers):
            dvec = c2 - lax.slice(c2, (0, jj), (c2.shape[0], jj + 1))
            sqd = jnp.sum(dvec * dvec, axis=0, keepdims=True)
            r_i += jnp.sum(jnp.maximum(_DELTA_R - jnp.sqrt(sqd), 0.0)) - _DELTA_R
        g_i = jnp.sum(jnp.sqrt(cn2))
        subl = lax.broadcasted_iota(jnp.int32, (k_centers, 128), 0)
        lanes = lax.broadcasted_iota(jnp.int32, (k_centers, 128), 1)
        vec = (jnp.where(lanes == 0, attr_ref[:, :1], 0.0)
               + jnp.where((lanes == 1) & (subl == 0), r_i, 0.0)
               + jnp.where((lanes == 2) & (subl == 0), g_i, 0.0))
        res_ref[0] = vec


def _tc_main(out_r, tgt_r, c2_r, cn2_r, lab_r, bn):
    b, d, hw = out_r.shape
    k = c2_r.shape[2]
    nb = hw // bn
    return pl.pallas_call(
        _tc_body,
        grid=(b, nb),
        in_specs=[
            pl.BlockSpec((1, d, bn), lambda i, j: (i, 0, j)),
            pl.BlockSpec((1, 1, bn), lambda i, j: (i, 0, j)),
            pl.BlockSpec((1, d, k), lambda i, j: (i, 0, 0)),
            pl.BlockSpec((1, k, 1), lambda i, j: (i, 0, 0)),
            pl.BlockSpec((1, k, 1), lambda i, j: (i, 0, 0)),
        ],
        out_specs=pl.BlockSpec((1, k, 128), lambda i, j: (i, 0, 0)),
        out_shape=jax.ShapeDtypeStruct((b, k, 128), jnp.float32),
        scratch_shapes=[
            pltpu.VMEM((k, 128), jnp.float32),
        ],
        compiler_params=pltpu.CompilerParams(
            dimension_semantics=("arbitrary", "arbitrary"),
        ),
    )(out_r, tgt_r, c2_r, cn2_r, lab_r)


def _assemble(res, counts, b, k):
    attr_raw = res[:, :, 0]
    r = res[:, 0, 1]
    g = res[:, 0, 2]
    n = jnp.sum(counts.reshape(b, k, 16), axis=2).astype(jnp.float32)
    denom = jnp.where(n > 1.0, n - 1.0, jnp.maximum(n, 1.0))
    a = jnp.sum(attr_raw / denom, axis=1)
    att = jnp.float32(0.0)
    rep = jnp.float32(0.0)
    reg = jnp.float32(0.0)
    for i in range(b):
        att = (att + a[i]) / k
        rep = (rep + r[i]) / (k * (k - 1))
        reg = (reg + g[i]) / k
    loss = _ALPHA * att + _BETA * rep + _GAMMA * reg
    return (loss, att, rep)


def kernel(out, target, centers, batch_size, device):
    b, d, h, w = out.shape
    k = centers.shape[1]
    hw = h * w

    centers = centers.astype(jnp.int32)
    target = target.astype(jnp.int32)
    p = centers[..., 0] * w + centers[..., 1]
    bidx = jnp.arange(b, dtype=jnp.int32)[:, None]
    lab_idx = (bidx * hw + p).reshape(-1)
    lab_idx_rep = jnp.broadcast_to(lab_idx[:, None], (b * k, 16))

    counts = _sc_counts(target.reshape(-1), lab_idx_rep, b, k, hw)

    p_flat = p.reshape(-1)
    c2_r, cn2_g, lab_g = _tc_gather(out.reshape(b, d, hw),
                                    target.reshape(b, 1, hw),
                                    p_flat // 128, p_flat % 128)
    cn2_r = cn2_g.reshape(b, k, 1)
    lab_r = lab_g.reshape(b, k, 1)

    res = _tc_main(out.reshape(b, d, hw), target.reshape(b, 1, hw),
                   c2_r, cn2_r, lab_r, bn=3584)
    return _assemble(res, counts, b, k)


# speedup vs baseline: 1.0890x; 1.0890x over previous
import functools

import jax
import jax.numpy as jnp
from jax import lax
from jax.experimental import pallas as pl
from jax.experimental.pallas import tpu as pltpu
from jax.experimental.pallas import tpu_sc as plsc

_DELTA_A = 0.1
_DELTA_R = 1.0
_ALPHA = 1.0
_BETA = 1.0
_GAMMA = 0.001


def _sc_counts(tgt_flat, lab_idx_rep, b, k, hw):
    bk = b * k
    info = plsc.get_sparse_core_info()
    nw = info.num_cores * info.num_subcores
    pairs = bk // nw
    nvec = hw // 16

    @functools.partial(
        pl.kernel,
        mesh=plsc.VectorSubcoreMesh(core_axis_name="c", subcore_axis_name="s"),
        out_type=jax.ShapeDtypeStruct((bk, 16), jnp.int32),
        scratch_types=[
            pltpu.VMEM((hw,), jnp.int32),
            pltpu.VMEM((pairs, 16), jnp.int32),
            pltpu.VMEM((pairs, 16), jnp.int32),
            pltpu.VMEM((pairs, 16), jnp.int32),
            pltpu.SemaphoreType.DMA,
        ],
    )
    def count_kernel(tgt_hbm, lidx_hbm, cnt_out, img_v, labs_v, lidx_v,
                     acc_v, sem):
        wid = lax.axis_index("s") * info.num_cores + lax.axis_index("c")
        img = (wid * pairs) // k
        pltpu.sync_copy(lidx_hbm.at[pl.ds(wid * pairs, pairs)], lidx_v)
        for q in range(pairs):
            pltpu.async_copy(tgt_hbm.at[lidx_v.at[q]], labs_v.at[q], sem).wait()
        pltpu.sync_copy(tgt_hbm.at[pl.ds(img * hw, hw)], img_v)
        splats = [labs_v[q, :] for q in range(pairs)]

        def step(it, accs):
            tv = img_v[pl.ds(it * 16, 16)]
            return tuple(
                acc + jnp.where(tv == splats[q], jnp.int32(1), jnp.int32(0))
                for q, acc in enumerate(accs)
            )

        accs = lax.fori_loop(0, nvec, step,
                             tuple(jnp.zeros((16,), jnp.int32)
                                   for _ in range(pairs)))
        for q in range(pairs):
            acc_v[q, :] = accs[q]
        pltpu.sync_copy(acc_v, cnt_out.at[pl.ds(wid * pairs, pairs)])

    return count_kernel(tgt_flat, lab_idx_rep)


def _make_tc_gather_body(k):
    def body(blk_ref, off_ref, *refs):
        o_refs = refs[:k]
        t_refs = refs[k:2 * k]
        c_ref, cn2_ref, lab_ref = refs[2 * k:]
        i = pl.program_id(0)
        lanes = lax.broadcasted_iota(jnp.int32, (1, 128), 1)
        lanes_k = lax.broadcasted_iota(jnp.int32, (1, k), 1)
        cn2row = jnp.zeros((1, k), jnp.float32)
        labrow = jnp.zeros((1, k), jnp.int32)
        for j in range(k):
            po = off_ref[i * k + j]
            mskf = (lanes == po).astype(jnp.float32)
            col = jnp.sum(o_refs[j][0] * mskf, axis=1, keepdims=True)
            c_ref[0, :, j:j + 1] = col
            cn2row += jnp.sum(col * col) * (lanes_k == j).astype(jnp.float32)
            labv = jnp.sum(t_refs[j][0] * (lanes == po).astype(jnp.int32),
                           axis=1, keepdims=True)
            labrow += labv * (lanes_k == j).astype(jnp.int32)
        cn2_ref[0] = cn2row
        lab_ref[0] = labrow
    return body


def _tc_gather(out_r, tgt_r, blk, off):
    b, d, hw = out_r.shape
    bk = blk.shape[0]
    k = bk // b

    def mk_in(j):
        return pl.BlockSpec((1, d, 128),
                            lambda i, blk, off, j=j: (i, 0, blk[i * k + j]))

    def mk_tin(j):
        return pl.BlockSpec((1, 1, 128),
                            lambda i, blk, off, j=j: (i, 0, blk[i * k + j]))

    grid_spec = pltpu.PrefetchScalarGridSpec(
        num_scalar_prefetch=2,
        grid=(b,),
        in_specs=([mk_in(j) for j in range(k)]
                  + [mk_tin(j) for j in range(k)]),
        out_specs=[
            pl.BlockSpec((1, d, k), lambda i, blk, off: (i, 0, 0)),
            pl.BlockSpec((1, 1, k), lambda i, blk, off: (i, 0, 0)),
            pl.BlockSpec((1, 1, k), lambda i, blk, off: (i, 0, 0)),
        ],
    )
    c2, cn2, lab = pl.pallas_call(
        _make_tc_gather_body(k),
        grid_spec=grid_spec,
        out_shape=[
            jax.ShapeDtypeStruct((b, d, k), jnp.float32),
            jax.ShapeDtypeStruct((b, 1, k), jnp.float32),
            jax.ShapeDtypeStruct((b, 1, k), jnp.int32),
        ],
    )(blk, off, *([out_r] * k), *([tgt_r] * k))
    return c2, cn2, lab


def _tc_body(out_ref, tgt_ref, c_ref, cn2_ref, lab_ref, res_ref, attr_ref):
    j = pl.program_id(1)
    nb = pl.num_programs(1)
    o = out_ref[0]
    t = tgt_ref[0]
    c2 = c_ref[0]
    cn2 = cn2_ref[0]
    lab = lab_ref[0]
    k_centers = c2.shape[1]

    @pl.when(j == 0)
    def _():
        attr_ref[...] = jnp.zeros_like(attr_ref)

    g = lax.dot_general(c2, o, (((0,), (0,)), ((), ())),
                        preferred_element_type=jnp.float32,
                        precision=lax.Precision.DEFAULT)
    pn2 = jnp.sum(o * o, axis=0, keepdims=True)
    sq = jnp.maximum(cn2 + pn2 - 2.0 * g, 0.0)
    hinged = jnp.maximum(jnp.sqrt(sq) - _DELTA_A, 0.0)
    hm = jnp.where(t == lab, hinged, 0.0)
    attr_ref[:, :1] += jnp.sum(hm, axis=1, keepdims=True)

    @pl.when(j == nb - 1)
    def _():
        r_i = jnp.float32(0.0)
        for jj in range(k_centers):
            dvec = c2 - lax.slice(c2, (0, jj), (c2.shape[0], jj + 1))
            sqd = jnp.sum(dvec * dvec, axis=0, keepdims=True)
            r_i += jnp.sum(jnp.maximum(_DELTA_R - jnp.sqrt(sqd), 0.0)) - _DELTA_R
        g_i = jnp.sum(jnp.sqrt(cn2))
        subl = lax.broadcasted_iota(jnp.int32, (k_centers, 128), 0)
        lanes = lax.broadcasted_iota(jnp.int32, (k_centers, 128), 1)
        vec = (jnp.where(lanes == 0, attr_ref[:, :1], 0.0)
               + jnp.where((lanes == 1) & (subl == 0), r_i, 0.0)
               + jnp.where((lanes == 2) & (subl == 0), g_i, 0.0))
        res_ref[0] = vec


def _tc_main(out_r, tgt_r, c2_r, cn2_r, lab_r, bn):
    b, d, hw = out_r.shape
    k = c2_r.shape[2]
    nb = hw // bn
    return pl.pallas_call(
        _tc_body,
        grid=(b, nb),
        in_specs=[
            pl.BlockSpec((1, d, bn), lambda i, j: (i, 0, j)),
            pl.BlockSpec((1, 1, bn), lambda i, j: (i, 0, j)),
            pl.BlockSpec((1, d, k), lambda i, j: (i, 0, 0)),
            pl.BlockSpec((1, k, 1), lambda i, j: (i, 0, 0)),
            pl.BlockSpec((1, k, 1), lambda i, j: (i, 0, 0)),
        ],
        out_specs=pl.BlockSpec((1, k, 128), lambda i, j: (i, 0, 0)),
        out_shape=jax.ShapeDtypeStruct((b, k, 128), jnp.float32),
        scratch_shapes=[
            pltpu.VMEM((k, 128), jnp.float32),
        ],
        compiler_params=pltpu.CompilerParams(
            dimension_semantics=("arbitrary", "arbitrary"),
        ),
    )(out_r, tgt_r, c2_r, cn2_r, lab_r)


def _assemble(res, counts, b, k):
    attr_raw = res[:, :, 0]
    r = res[:, 0, 1]
    g = res[:, 0, 2]
    n = jnp.sum(counts.reshape(b, k, 16), axis=2).astype(jnp.float32)
    denom = jnp.where(n > 1.0, n - 1.0, jnp.maximum(n, 1.0))
    a = jnp.sum(attr_raw / denom, axis=1)
    att = jnp.float32(0.0)
    rep = jnp.float32(0.0)
    reg = jnp.float32(0.0)
    for i in range(b):
        att = (att + a[i]) / k
        rep = (rep + r[i]) / (k * (k - 1))
        reg = (reg + g[i]) / k
    loss = _ALPHA * att + _BETA * rep + _GAMMA * reg
    return (loss, att, rep)


def kernel(out, target, centers, batch_size, device):
    b, d, h, w = out.shape
    k = centers.shape[1]
    hw = h * w

    centers = centers.astype(jnp.int32)
    target = target.astype(jnp.int32)
    p = centers[..., 0] * w + centers[..., 1]
    bidx = jnp.arange(b, dtype=jnp.int32)[:, None]
    lab_idx = (bidx * hw + p).reshape(-1)
    lab_idx_rep = jnp.broadcast_to(lab_idx[:, None], (b * k, 16))

    counts = _sc_counts(target.reshape(-1), lab_idx_rep, b, k, hw)

    p_flat = p.reshape(-1)
    c2_r, cn2_g, lab_g = _tc_gather(out.reshape(b, d, hw),
                                    target.reshape(b, 1, hw),
                                    p_flat // 128, p_flat % 128)
    cn2_r = cn2_g.reshape(b, k, 1)
    lab_r = lab_g.reshape(b, k, 1)

    res = _tc_main(out.reshape(b, d, hw), target.reshape(b, 1, hw),
                   c2_r, cn2_r, lab_r, bn=7168)
    return _assemble(res, counts, b, k)


# speedup vs baseline: 1.1246x; 1.0327x over previous
import functools

import jax
import jax.numpy as jnp
from jax import lax
from jax.experimental import pallas as pl
from jax.experimental.pallas import tpu as pltpu
from jax.experimental.pallas import tpu_sc as plsc

_DELTA_A = 0.1
_DELTA_R = 1.0
_ALPHA = 1.0
_BETA = 1.0
_GAMMA = 0.001


def _sc_counts(tgt_flat, lab_idx_rep, b, k, hw):
    bk = b * k
    info = plsc.get_sparse_core_info()
    nw = info.num_cores * info.num_subcores
    pairs = bk // nw
    nvec = hw // 16

    @functools.partial(
        pl.kernel,
        mesh=plsc.VectorSubcoreMesh(core_axis_name="c", subcore_axis_name="s"),
        out_type=jax.ShapeDtypeStruct((bk, 16), jnp.int32),
        scratch_types=[
            pltpu.VMEM((hw,), jnp.int32),
            pltpu.VMEM((pairs, 16), jnp.int32),
            pltpu.VMEM((pairs, 16), jnp.int32),
            pltpu.VMEM((pairs, 16), jnp.int32),
            pltpu.SemaphoreType.DMA,
        ],
    )
    def count_kernel(tgt_hbm, lidx_hbm, cnt_out, img_v, labs_v, lidx_v,
                     acc_v, sem):
        wid = lax.axis_index("s") * info.num_cores + lax.axis_index("c")
        img = (wid * pairs) // k
        pltpu.sync_copy(lidx_hbm.at[pl.ds(wid * pairs, pairs)], lidx_v)
        for q in range(pairs):
            pltpu.async_copy(tgt_hbm.at[lidx_v.at[q]], labs_v.at[q], sem).wait()
        pltpu.sync_copy(tgt_hbm.at[pl.ds(img * hw, hw)], img_v)
        splats = [labs_v[q, :] for q in range(pairs)]

        def step(it, accs):
            tv = img_v[pl.ds(it * 16, 16)]
            return tuple(
                acc + jnp.where(tv == splats[q], jnp.int32(1), jnp.int32(0))
                for q, acc in enumerate(accs)
            )

        accs = lax.fori_loop(0, nvec, step,
                             tuple(jnp.zeros((16,), jnp.int32)
                                   for _ in range(pairs)))
        for q in range(pairs):
            acc_v[q, :] = accs[q]
        pltpu.sync_copy(acc_v, cnt_out.at[pl.ds(wid * pairs, pairs)])

    return count_kernel(tgt_flat, lab_idx_rep)


def _make_tc_gather_body(k):
    def body(blk_ref, off_ref, *refs):
        o_refs = refs[:k]
        t_refs = refs[k:2 * k]
        c_ref, cn2_ref, lab_ref = refs[2 * k:]
        i = pl.program_id(0)
        lanes = lax.broadcasted_iota(jnp.int32, (1, 128), 1)
        lanes_k = lax.broadcasted_iota(jnp.int32, (1, k), 1)
        cn2row = jnp.zeros((1, k), jnp.float32)
        labrow = jnp.zeros((1, k), jnp.int32)
        for j in range(k):
            po = off_ref[i * k + j]
            mskf = (lanes == po).astype(jnp.float32)
            col = jnp.sum(o_refs[j][0] * mskf, axis=1, keepdims=True)
            c_ref[0, :, j:j + 1] = col
            cn2row += jnp.sum(col * col) * (lanes_k == j).astype(jnp.float32)
            labv = jnp.sum(t_refs[j][0] * (lanes == po).astype(jnp.int32),
                           axis=1, keepdims=True)
            labrow += labv * (lanes_k == j).astype(jnp.int32)
        cn2_ref[0] = cn2row
        lab_ref[0] = labrow
    return body


def _tc_gather(out_r, tgt_r, blk, off):
    b, d, hw = out_r.shape
    bk = blk.shape[0]
    k = bk // b

    def mk_in(j):
        return pl.BlockSpec((1, d, 128),
                            lambda i, blk, off, j=j: (i, 0, blk[i * k + j]))

    def mk_tin(j):
        return pl.BlockSpec((1, 1, 128),
                            lambda i, blk, off, j=j: (i, 0, blk[i * k + j]))

    grid_spec = pltpu.PrefetchScalarGridSpec(
        num_scalar_prefetch=2,
        grid=(b,),
        in_specs=([mk_in(j) for j in range(k)]
                  + [mk_tin(j) for j in range(k)]),
        out_specs=[
            pl.BlockSpec((1, d, k), lambda i, blk, off: (i, 0, 0)),
            pl.BlockSpec((1, 1, k), lambda i, blk, off: (i, 0, 0)),
            pl.BlockSpec((1, 1, k), lambda i, blk, off: (i, 0, 0)),
        ],
    )
    c2, cn2, lab = pl.pallas_call(
        _make_tc_gather_body(k),
        grid_spec=grid_spec,
        out_shape=[
            jax.ShapeDtypeStruct((b, d, k), jnp.float32),
            jax.ShapeDtypeStruct((b, 1, k), jnp.float32),
            jax.ShapeDtypeStruct((b, 1, k), jnp.int32),
        ],
    )(blk, off, *([out_r] * k), *([tgt_r] * k))
    return c2, cn2, lab


def _tc_body(out_ref, tgt_ref, c_ref, cn2_ref, lab_ref, res_ref, attr_ref):
    j = pl.program_id(1)
    nb = pl.num_programs(1)
    o = out_ref[0]
    t = tgt_ref[0]
    c2 = c_ref[0]
    cn2 = cn2_ref[0]
    lab = lab_ref[0]
    k_centers = c2.shape[1]

    @pl.when(j == 0)
    def _():
        attr_ref[...] = jnp.zeros_like(attr_ref)

    g = lax.dot_general(c2, o, (((0,), (0,)), ((), ())),
                        preferred_element_type=jnp.float32,
                        precision=lax.Precision.DEFAULT)
    pn2 = jnp.sum(o * o, axis=0, keepdims=True)
    sq = jnp.maximum(cn2 + pn2 - 2.0 * g, 0.0)
    hinged = jnp.maximum(jnp.sqrt(sq) - _DELTA_A, 0.0)
    hm = jnp.where(t == lab, hinged, 0.0)
    attr_ref[:, :1] += jnp.sum(hm, axis=1, keepdims=True)

    @pl.when(j == nb - 1)
    def _():
        r_i = jnp.float32(0.0)
        for jj in range(k_centers):
            dvec = c2 - lax.slice(c2, (0, jj), (c2.shape[0], jj + 1))
            sqd = jnp.sum(dvec * dvec, axis=0, keepdims=True)
            r_i += jnp.sum(jnp.maximum(_DELTA_R - jnp.sqrt(sqd), 0.0)) - _DELTA_R
        g_i = jnp.sum(jnp.sqrt(cn2))
        subl = lax.broadcasted_iota(jnp.int32, (k_centers, 128), 0)
        lanes = lax.broadcasted_iota(jnp.int32, (k_centers, 128), 1)
        vec = (jnp.where(lanes == 0, attr_ref[:, :1], 0.0)
               + jnp.where((lanes == 1) & (subl == 0), r_i, 0.0)
               + jnp.where((lanes == 2) & (subl == 0), g_i, 0.0))
        res_ref[0] = vec


def _tc_main(out_r, tgt_r, c2_r, cn2_r, lab_r, bn):
    b, d, hw = out_r.shape
    k = c2_r.shape[2]
    nb = hw // bn
    return pl.pallas_call(
        _tc_body,
        grid=(b, nb),
        in_specs=[
            pl.BlockSpec((1, d, bn), lambda i, j: (i, 0, j)),
            pl.BlockSpec((1, 1, bn), lambda i, j: (i, 0, j)),
            pl.BlockSpec((1, d, k), lambda i, j: (i, 0, 0)),
            pl.BlockSpec((1, k, 1), lambda i, j: (i, 0, 0)),
            pl.BlockSpec((1, k, 1), lambda i, j: (i, 0, 0)),
        ],
        out_specs=pl.BlockSpec((1, k, 128), lambda i, j: (i, 0, 0)),
        out_shape=jax.ShapeDtypeStruct((b, k, 128), jnp.float32),
        scratch_shapes=[
            pltpu.VMEM((k, 128), jnp.float32),
        ],
        compiler_params=pltpu.CompilerParams(
            dimension_semantics=("arbitrary", "arbitrary"),
        ),
    )(out_r, tgt_r, c2_r, cn2_r, lab_r)


def _assemble(res, counts, b, k):
    attr_raw = res[:, :, 0]
    r = res[:, 0, 1]
    g = res[:, 0, 2]
    n = jnp.sum(counts.reshape(b, k, 16), axis=2).astype(jnp.float32)
    denom = jnp.where(n > 1.0, n - 1.0, jnp.maximum(n, 1.0))
    a = jnp.sum(attr_raw / denom, axis=1)
    att = jnp.float32(0.0)
    rep = jnp.float32(0.0)
    reg = jnp.float32(0.0)
    for i in range(b):
        att = (att + a[i]) / k
        rep = (rep + r[i]) / (k * (k - 1))
        reg = (reg + g[i]) / k
    loss = _ALPHA * att + _BETA * rep + _GAMMA * reg
    return (loss, att, rep)


def kernel(out, target, centers, batch_size, device):
    b, d, h, w = out.shape
    k = centers.shape[1]
    hw = h * w

    centers = centers.astype(jnp.int32)
    target = target.astype(jnp.int32)
    p = centers[..., 0] * w + centers[..., 1]
    bidx = jnp.arange(b, dtype=jnp.int32)[:, None]
    lab_idx = (bidx * hw + p).reshape(-1)
    lab_idx_rep = jnp.broadcast_to(lab_idx[:, None], (b * k, 16))

    counts = _sc_counts(target.reshape(-1), lab_idx_rep, b, k, hw)

    p_flat = p.reshape(-1)
    c2_r, cn2_g, lab_g = _tc_gather(out.reshape(b, d, hw),
                                    target.reshape(b, 1, hw),
                                    p_flat // 128, p_flat % 128)
    cn2_r = cn2_g.reshape(b, k, 1)
    lab_r = lab_g.reshape(b, k, 1)

    res = _tc_main(out.reshape(b, d, hw), target.reshape(b, 1, hw),
                   c2_r, cn2_r, lab_r, bn=12544)
    return _assemble(res, counts, b, k)


# speedup vs baseline: 1.1484x; 1.0212x over previous
import functools

import jax
import jax.numpy as jnp
from jax import lax
from jax.experimental import pallas as pl
from jax.experimental.pallas import tpu as pltpu
from jax.experimental.pallas import tpu_sc as plsc

_DELTA_A = 0.1
_DELTA_R = 1.0
_ALPHA = 1.0
_BETA = 1.0
_GAMMA = 0.001


def _sc_counts(tgt_flat, lab_idx_rep, b, k, hw):
    bk = b * k
    info = plsc.get_sparse_core_info()
    nw = info.num_cores * info.num_subcores
    pairs = bk // nw
    nvec = hw // 16

    @functools.partial(
        pl.kernel,
        mesh=plsc.VectorSubcoreMesh(core_axis_name="c", subcore_axis_name="s"),
        out_type=jax.ShapeDtypeStruct((bk, 16), jnp.int32),
        scratch_types=[
            pltpu.VMEM((hw,), jnp.int32),
            pltpu.VMEM((pairs, 16), jnp.int32),
            pltpu.VMEM((pairs, 16), jnp.int32),
            pltpu.VMEM((pairs, 16), jnp.int32),
            pltpu.SemaphoreType.DMA,
        ],
    )
    def count_kernel(tgt_hbm, lidx_hbm, cnt_out, img_v, labs_v, lidx_v,
                     acc_v, sem):
        wid = lax.axis_index("s") * info.num_cores + lax.axis_index("c")
        img = (wid * pairs) // k
        pltpu.sync_copy(lidx_hbm.at[pl.ds(wid * pairs, pairs)], lidx_v)
        for q in range(pairs):
            pltpu.async_copy(tgt_hbm.at[lidx_v.at[q]], labs_v.at[q], sem).wait()
        pltpu.sync_copy(tgt_hbm.at[pl.ds(img * hw, hw)], img_v)
        splats = [labs_v[q, :] for q in range(pairs)]

        def step(it, accs):
            tv = img_v[pl.ds(it * 16, 16)]
            return tuple(
                acc + jnp.where(tv == splats[q], jnp.int32(1), jnp.int32(0))
                for q, acc in enumerate(accs)
            )

        accs = lax.fori_loop(0, nvec, step,
                             tuple(jnp.zeros((16,), jnp.int32)
                                   for _ in range(pairs)))
        for q in range(pairs):
            acc_v[q, :] = accs[q]
        pltpu.sync_copy(acc_v, cnt_out.at[pl.ds(wid * pairs, pairs)])

    return count_kernel(tgt_flat, lab_idx_rep)


def _make_tc_gather_body(k):
    def body(blk_ref, off_ref, *refs):
        o_refs = refs[:k]
        t_refs = refs[k:2 * k]
        c_ref, cn2_ref, lab_ref = refs[2 * k:]
        i = pl.program_id(0)
        lanes = lax.broadcasted_iota(jnp.int32, (1, 128), 1)
        lanes_k = lax.broadcasted_iota(jnp.int32, (1, k), 1)
        cn2row = jnp.zeros((1, k), jnp.float32)
        labrow = jnp.zeros((1, k), jnp.int32)
        for j in range(k):
            po = off_ref[i * k + j]
            mskf = (lanes == po).astype(jnp.float32)
            col = jnp.sum(o_refs[j][0] * mskf, axis=1, keepdims=True)
            c_ref[0, :, j:j + 1] = col
            cn2row += jnp.sum(col * col) * (lanes_k == j).astype(jnp.float32)
            labv = jnp.sum(t_refs[j][0] * (lanes == po).astype(jnp.int32),
                           axis=1, keepdims=True)
            labrow += labv * (lanes_k == j).astype(jnp.int32)
        cn2_ref[0] = cn2row
        lab_ref[0] = labrow
    return body


def _tc_gather(out_r, tgt_r, blk, off):
    b, d, hw = out_r.shape
    bk = blk.shape[0]
    k = bk // b

    def mk_in(j):
        return pl.BlockSpec((1, d, 128),
                            lambda i, blk, off, j=j: (i, 0, blk[i * k + j]))

    def mk_tin(j):
        return pl.BlockSpec((1, 1, 128),
                            lambda i, blk, off, j=j: (i, 0, blk[i * k + j]))

    grid_spec = pltpu.PrefetchScalarGridSpec(
        num_scalar_prefetch=2,
        grid=(b,),
        in_specs=([mk_in(j) for j in range(k)]
                  + [mk_tin(j) for j in range(k)]),
        out_specs=[
            pl.BlockSpec((1, d, k), lambda i, blk, off: (i, 0, 0)),
            pl.BlockSpec((1, 1, k), lambda i, blk, off: (i, 0, 0)),
            pl.BlockSpec((1, 1, k), lambda i, blk, off: (i, 0, 0)),
        ],
    )
    c2, cn2, lab = pl.pallas_call(
        _make_tc_gather_body(k),
        grid_spec=grid_spec,
        out_shape=[
            jax.ShapeDtypeStruct((b, d, k), jnp.float32),
            jax.ShapeDtypeStruct((b, 1, k), jnp.float32),
            jax.ShapeDtypeStruct((b, 1, k), jnp.int32),
        ],
    )(blk, off, *([out_r] * k), *([tgt_r] * k))
    return c2, cn2, lab


def _tc_body(out_ref, tgt_ref, c_ref, cn2_ref, lab_ref, res_ref, attr_ref):
    j = pl.program_id(1)
    nb = pl.num_programs(1)
    o = out_ref[0]
    t = tgt_ref[0]
    c2 = c_ref[0]
    cn2 = cn2_ref[0]
    lab = lab_ref[0]
    k_centers = c2.shape[1]

    @pl.when(j == 0)
    def _():
        attr_ref[...] = jnp.zeros_like(attr_ref)

    g = lax.dot_general(c2, o, (((0,), (0,)), ((), ())),
                        preferred_element_type=jnp.float32,
                        precision=lax.Precision.DEFAULT)
    pn2 = jnp.sum(o * o, axis=0, keepdims=True)
    sq = jnp.maximum(cn2 + pn2 - 2.0 * g, 0.0)
    hinged = jnp.maximum(jnp.sqrt(sq) - _DELTA_A, 0.0)
    hm = jnp.where(t == lab, hinged, 0.0)
    attr_ref[:, :1] += jnp.sum(hm, axis=1, keepdims=True)

    @pl.when(j == nb - 1)
    def _():
        r_i = jnp.float32(0.0)
        for jj in range(k_centers):
            dvec = c2 - lax.slice(c2, (0, jj), (c2.shape[0], jj + 1))
            sqd = jnp.sum(dvec * dvec, axis=0, keepdims=True)
            r_i += jnp.sum(jnp.maximum(_DELTA_R - jnp.sqrt(sqd), 0.0)) - _DELTA_R
        g_i = jnp.sum(jnp.sqrt(cn2))
        subl = lax.broadcasted_iota(jnp.int32, (k_centers, 128), 0)
        lanes = lax.broadcasted_iota(jnp.int32, (k_centers, 128), 1)
        vec = (jnp.where(lanes == 0, attr_ref[:, :1], 0.0)
               + jnp.where((lanes == 1) & (subl == 0), r_i, 0.0)
               + jnp.where((lanes == 2) & (subl == 0), g_i, 0.0))
        res_ref[0] = vec


def _tc_main(out_r, tgt_r, c2_r, cn2_r, lab_r, bn):
    b, d, hw = out_r.shape
    k = c2_r.shape[2]
    nb = hw // bn
    return pl.pallas_call(
        _tc_body,
        grid=(b, nb),
        in_specs=[
            pl.BlockSpec((1, d, bn), lambda i, j: (i, 0, j)),
            pl.BlockSpec((1, 1, bn), lambda i, j: (i, 0, j)),
            pl.BlockSpec((1, d, k), lambda i, j: (i, 0, 0)),
            pl.BlockSpec((1, k, 1), lambda i, j: (i, 0, 0)),
            pl.BlockSpec((1, k, 1), lambda i, j: (i, 0, 0)),
        ],
        out_specs=pl.BlockSpec((1, k, 128), lambda i, j: (i, 0, 0)),
        out_shape=jax.ShapeDtypeStruct((b, k, 128), jnp.float32),
        scratch_shapes=[
            pltpu.VMEM((k, 128), jnp.float32),
        ],
        compiler_params=pltpu.CompilerParams(
            dimension_semantics=("arbitrary", "arbitrary"),
        ),
    )(out_r, tgt_r, c2_r, cn2_r, lab_r)


def _assemble(res, counts, b, k):
    attr_raw = res[:, :, 0]
    r = res[:, 0, 1]
    g = res[:, 0, 2]
    n = jnp.sum(counts.reshape(b, k, 16), axis=2).astype(jnp.float32)
    denom = jnp.where(n > 1.0, n - 1.0, jnp.maximum(n, 1.0))
    a = jnp.sum(attr_raw / denom, axis=1)
    att = jnp.float32(0.0)
    rep = jnp.float32(0.0)
    reg = jnp.float32(0.0)
    for i in range(b):
        att = (att + a[i]) / k
        rep = (rep + r[i]) / (k * (k - 1))
        reg = (reg + g[i]) / k
    loss = _ALPHA * att + _BETA * rep + _GAMMA * reg
    return (loss, att, rep)


def kernel(out, target, centers, batch_size, device):
    b, d, h, w = out.shape
    k = centers.shape[1]
    hw = h * w

    centers = centers.astype(jnp.int32)
    target = target.astype(jnp.int32)
    p = centers[..., 0] * w + centers[..., 1]
    bidx = jnp.arange(b, dtype=jnp.int32)[:, None]
    lab_idx = (bidx * hw + p).reshape(-1)
    lab_idx_rep = jnp.broadcast_to(lab_idx[:, None], (b * k, 16))

    counts = _sc_counts(target.reshape(-1), lab_idx_rep, b, k, hw)

    p_flat = p.reshape(-1)
    c2_r, cn2_g, lab_g = _tc_gather(out.reshape(b, d, hw),
                                    target.reshape(b, 1, hw),
                                    p_flat // 128, p_flat % 128)
    cn2_r = cn2_g.reshape(b, k, 1)
    lab_r = lab_g.reshape(b, k, 1)

    res = _tc_main(out.reshape(b, d, hw), target.reshape(b, 1, hw),
                   c2_r, cn2_r, lab_r, bn=25088)
    return _assemble(res, counts, b, k)


# speedup vs baseline: 1.1961x; 1.0415x over previous
import functools

import jax
import jax.numpy as jnp
from jax import lax
from jax.experimental import pallas as pl
from jax.experimental.pallas import tpu as pltpu
from jax.experimental.pallas import tpu_sc as plsc

_DELTA_A = 0.1
_DELTA_R = 1.0
_ALPHA = 1.0
_BETA = 1.0
_GAMMA = 0.001


def _sc_counts(tgt_flat, lab_idx_rep, b, k, hw):
    bk = b * k
    info = plsc.get_sparse_core_info()
    nw = info.num_cores * info.num_subcores
    pairs = bk // nw
    nvec = hw // 16

    @functools.partial(
        pl.kernel,
        mesh=plsc.VectorSubcoreMesh(core_axis_name="c", subcore_axis_name="s"),
        out_type=jax.ShapeDtypeStruct((bk, 16), jnp.int32),
        scratch_types=[
            pltpu.VMEM((hw,), jnp.int32),
            pltpu.VMEM((pairs, 16), jnp.int32),
            pltpu.VMEM((pairs, 16), jnp.int32),
            pltpu.VMEM((pairs, 16), jnp.int32),
            pltpu.SemaphoreType.DMA,
        ],
    )
    def count_kernel(tgt_hbm, lidx_hbm, cnt_out, img_v, labs_v, lidx_v,
                     acc_v, sem):
        wid = lax.axis_index("s") * info.num_cores + lax.axis_index("c")
        img = (wid * pairs) // k
        pltpu.sync_copy(lidx_hbm.at[pl.ds(wid * pairs, pairs)], lidx_v)
        for q in range(pairs):
            pltpu.async_copy(tgt_hbm.at[lidx_v.at[q]], labs_v.at[q], sem).wait()
        pltpu.sync_copy(tgt_hbm.at[pl.ds(img * hw, hw)], img_v)
        splats = [labs_v[q, :] for q in range(pairs)]

        def step(it, accs):
            tv = img_v[pl.ds(it * 16, 16)]
            return tuple(
                acc + jnp.where(tv == splats[q], jnp.int32(1), jnp.int32(0))
                for q, acc in enumerate(accs)
            )

        accs = lax.fori_loop(0, nvec, step,
                             tuple(jnp.zeros((16,), jnp.int32)
                                   for _ in range(pairs)))
        for q in range(pairs):
            acc_v[q, :] = accs[q]
        pltpu.sync_copy(acc_v, cnt_out.at[pl.ds(wid * pairs, pairs)])

    return count_kernel(tgt_flat, lab_idx_rep)


def _make_tc_gather_body(k):
    def body(blk_ref, off_ref, *refs):
        o_refs = refs[:k]
        t_refs = refs[k:2 * k]
        c_ref, cn2_ref, lab_ref = refs[2 * k:]
        i = pl.program_id(0)
        lanes = lax.broadcasted_iota(jnp.int32, (1, 128), 1)
        lanes_k = lax.broadcasted_iota(jnp.int32, (1, k), 1)
        cn2row = jnp.zeros((1, k), jnp.float32)
        labrow = jnp.zeros((1, k), jnp.int32)
        for j in range(k):
            po = off_ref[i * k + j]
            mskf = (lanes == po).astype(jnp.float32)
            col = jnp.sum(o_refs[j][0] * mskf, axis=1, keepdims=True)
            c_ref[0, :, j:j + 1] = col
            cn2row += jnp.sum(col * col) * (lanes_k == j).astype(jnp.float32)
            labv = jnp.sum(t_refs[j][0] * (lanes == po).astype(jnp.int32),
                           axis=1, keepdims=True)
            labrow += labv * (lanes_k == j).astype(jnp.int32)
        cn2_ref[0] = cn2row
        lab_ref[0] = labrow
    return body


def _tc_gather(out_r, tgt_r, blk, off):
    b, d, hw = out_r.shape
    bk = blk.shape[0]
    k = bk // b

    def mk_in(j):
        return pl.BlockSpec((1, d, 128),
                            lambda i, blk, off, j=j: (i, 0, blk[i * k + j]))

    def mk_tin(j):
        return pl.BlockSpec((1, 1, 128),
                            lambda i, blk, off, j=j: (i, 0, blk[i * k + j]))

    grid_spec = pltpu.PrefetchScalarGridSpec(
        num_scalar_prefetch=2,
        grid=(b,),
        in_specs=([mk_in(j) for j in range(k)]
                  + [mk_tin(j) for j in range(k)]),
        out_specs=[
            pl.BlockSpec((1, d, k), lambda i, blk, off: (i, 0, 0)),
            pl.BlockSpec((1, 1, k), lambda i, blk, off: (i, 0, 0)),
            pl.BlockSpec((1, 1, k), lambda i, blk, off: (i, 0, 0)),
        ],
    )
    c2, cn2, lab = pl.pallas_call(
        _make_tc_gather_body(k),
        grid_spec=grid_spec,
        out_shape=[
            jax.ShapeDtypeStruct((b, d, k), jnp.float32),
            jax.ShapeDtypeStruct((b, 1, k), jnp.float32),
            jax.ShapeDtypeStruct((b, 1, k), jnp.int32),
        ],
    )(blk, off, *([out_r] * k), *([tgt_r] * k))
    return c2, cn2, lab


def _tc_body(out_ref, tgt_ref, c_ref, cn2_ref, lab_ref, res_ref, attr_ref):
    j = pl.program_id(1)
    nb = pl.num_programs(1)
    o = out_ref[0]
    t = tgt_ref[0]
    c2 = c_ref[0]
    cn2 = cn2_ref[0]
    lab = lab_ref[0]
    k_centers = c2.shape[1]

    @pl.when(j == 0)
    def _():
        attr_ref[...] = jnp.zeros_like(attr_ref)

    g = lax.dot_general(c2, o, (((0,), (0,)), ((), ())),
                        preferred_element_type=jnp.float32,
                        precision=lax.Precision.DEFAULT)
    pn2 = jnp.sum(o * o, axis=0, keepdims=True)
    sq = jnp.maximum(cn2 + pn2 - 2.0 * g, 0.0)
    hinged = jnp.maximum(jnp.sqrt(sq) - _DELTA_A, 0.0)
    hm = jnp.where(t == lab, hinged, 0.0)
    attr_ref[:, :1] += jnp.sum(hm, axis=1, keepdims=True)

    @pl.when(j == nb - 1)
    def _():
        r_i = jnp.float32(0.0)
        for jj in range(k_centers):
            dvec = c2 - lax.slice(c2, (0, jj), (c2.shape[0], jj + 1))
            sqd = jnp.sum(dvec * dvec, axis=0, keepdims=True)
            r_i += jnp.sum(jnp.maximum(_DELTA_R - jnp.sqrt(sqd), 0.0)) - _DELTA_R
        g_i = jnp.sum(jnp.sqrt(cn2))
        subl = lax.broadcasted_iota(jnp.int32, (k_centers, 128), 0)
        lanes = lax.broadcasted_iota(jnp.int32, (k_centers, 128), 1)
        vec = (jnp.where(lanes == 0, attr_ref[:, :1], 0.0)
               + jnp.where((lanes == 1) & (subl == 0), r_i, 0.0)
               + jnp.where((lanes == 2) & (subl == 0), g_i, 0.0))
        res_ref[0] = vec


def _tc_main(out_r, tgt_r, c2_r, cn2_r, lab_r, bn):
    b, d, hw = out_r.shape
    k = c2_r.shape[2]
    nb = hw // bn
    return pl.pallas_call(
        _tc_body,
        grid=(b, nb),
        in_specs=[
            pl.BlockSpec((1, d, bn), lambda i, j: (i, 0, j)),
            pl.BlockSpec((1, 1, bn), lambda i, j: (i, 0, j)),
            pl.BlockSpec((1, d, k), lambda i, j: (i, 0, 0)),
            pl.BlockSpec((1, k, 1), lambda i, j: (i, 0, 0)),
            pl.BlockSpec((1, k, 1), lambda i, j: (i, 0, 0)),
        ],
        out_specs=pl.BlockSpec((1, k, 128), lambda i, j: (i, 0, 0)),
        out_shape=jax.ShapeDtypeStruct((b, k, 128), jnp.float32),
        scratch_shapes=[
            pltpu.VMEM((k, 128), jnp.float32),
        ],
        compiler_params=pltpu.CompilerParams(
            dimension_semantics=("arbitrary", "arbitrary"),
        ),
    )(out_r, tgt_r, c2_r, cn2_r, lab_r)


def _assemble(res, counts, b, k):
    attr_raw = res[:, :, 0]
    r = res[:, 0, 1]
    g = res[:, 0, 2]
    n = jnp.sum(counts.reshape(b, k, 16), axis=2).astype(jnp.float32)
    denom = jnp.where(n > 1.0, n - 1.0, jnp.maximum(n, 1.0))
    a = jnp.sum(attr_raw / denom, axis=1)
    att = jnp.float32(0.0)
    rep = jnp.float32(0.0)
    reg = jnp.float32(0.0)
    for i in range(b):
        att = (att + a[i]) / k
        rep = (rep + r[i]) / (k * (k - 1))
        reg = (reg + g[i]) / k
    loss = _ALPHA * att + _BETA * rep + _GAMMA * reg
    return (loss, att, rep)


def kernel(out, target, centers, batch_size, device):
    b, d, h, w = out.shape
    k = centers.shape[1]
    hw = h * w

    centers = centers.astype(jnp.int32)
    target = target.astype(jnp.int32)
    p = centers[..., 0] * w + centers[..., 1]
    bidx = jnp.arange(b, dtype=jnp.int32)[:, None]
    lab_idx = (bidx * hw + p).reshape(-1)
    lab_idx_rep = jnp.broadcast_to(lab_idx[:, None], (b * k, 16))

    counts = _sc_counts(target.reshape(-1), lab_idx_rep, b, k, hw)

    p_flat = p.reshape(-1)
    c2_r, cn2_g, lab_g = _tc_gather(out.reshape(b, d, hw),
                                    target.reshape(b, 1, hw),
                                    p_flat // 128, p_flat % 128)
    cn2_r = cn2_g.reshape(b, k, 1)
    lab_r = lab_g.reshape(b, k, 1)

    res = _tc_main(out.reshape(b, d, hw), target.reshape(b, 1, hw),
                   c2_r, cn2_r, lab_r, bn=50176)
    return _assemble(res, counts, b, k)


# speedup vs baseline: 1.2560x; 1.0501x over previous
import functools

import jax
import jax.numpy as jnp
from jax import lax
from jax.experimental import pallas as pl
from jax.experimental.pallas import tpu as pltpu
from jax.experimental.pallas import tpu_sc as plsc

_DELTA_A = 0.1
_DELTA_R = 1.0
_ALPHA = 1.0
_BETA = 1.0
_GAMMA = 0.001


def _sc_counts(tgt_flat, lab_idx_rep, b, k, hw):
    bk = b * k
    info = plsc.get_sparse_core_info()
    nw = info.num_cores * info.num_subcores
    pairs = bk // nw
    nvec = hw // 16

    @functools.partial(
        pl.kernel,
        mesh=plsc.VectorSubcoreMesh(core_axis_name="c", subcore_axis_name="s"),
        out_type=jax.ShapeDtypeStruct((bk, 16), jnp.int32),
        scratch_types=[
            pltpu.VMEM((hw,), jnp.int32),
            pltpu.VMEM((pairs, 16), jnp.int32),
            pltpu.VMEM((pairs, 16), jnp.int32),
            pltpu.VMEM((pairs, 16), jnp.int32),
            pltpu.SemaphoreType.DMA,
        ],
    )
    def count_kernel(tgt_hbm, lidx_hbm, cnt_out, img_v, labs_v, lidx_v,
                     acc_v, sem):
        wid = lax.axis_index("s") * info.num_cores + lax.axis_index("c")
        img = (wid * pairs) // k
        pltpu.sync_copy(lidx_hbm.at[pl.ds(wid * pairs, pairs)], lidx_v)
        for q in range(pairs):
            pltpu.async_copy(tgt_hbm.at[lidx_v.at[q]], labs_v.at[q], sem).wait()
        pltpu.sync_copy(tgt_hbm.at[pl.ds(img * hw, hw)], img_v)
        splats = [labs_v[q, :] for q in range(pairs)]

        def step(it, accs):
            tv = img_v[pl.ds(it * 16, 16)]
            return tuple(
                acc + jnp.where(tv == splats[q], jnp.int32(1), jnp.int32(0))
                for q, acc in enumerate(accs)
            )

        accs = lax.fori_loop(0, nvec, step,
                             tuple(jnp.zeros((16,), jnp.int32)
                                   for _ in range(pairs)))
        for q in range(pairs):
            acc_v[q, :] = accs[q]
        pltpu.sync_copy(acc_v, cnt_out.at[pl.ds(wid * pairs, pairs)])

    return count_kernel(tgt_flat, lab_idx_rep)


def _make_tc_body(k):
    def body(blk_ref, off_ref, *refs):
        o_refs = refs[:k]
        t_refs = refs[k:2 * k]
        out_ref, tgt_ref, res_ref = refs[2 * k:]
        i = pl.program_id(0)
        o = out_ref[0]
        t = tgt_ref[0]
        dsz = o.shape[0]

        lanes128 = lax.broadcasted_iota(jnp.int32, (1, 128), 1)
        lanes_k = lax.broadcasted_iota(jnp.int32, (1, k), 1)
        subl_k = lax.broadcasted_iota(jnp.int32, (k, 1), 0)
        c2 = jnp.zeros((dsz, k), jnp.float32)
        cn2 = jnp.zeros((k, 1), jnp.float32)
        lab = jnp.zeros((k, 1), jnp.int32)
        for j in range(k):
            po = off_ref[i * k + j]
            mskf = (lanes128 == po).astype(jnp.float32)
            col = jnp.sum(o_refs[j][0] * mskf, axis=1, keepdims=True)
            c2 = c2 + col * (lanes_k == j).astype(jnp.float32)
            cn2 = cn2 + jnp.sum(col * col) * (subl_k == j).astype(jnp.float32)
            labv = jnp.sum(t_refs[j][0] * (lanes128 == po).astype(jnp.int32),
                           axis=1, keepdims=True)
            lab = lab + labv * (subl_k == j).astype(jnp.int32)

        g = lax.dot_general(c2, o, (((0,), (0,)), ((), ())),
                            preferred_element_type=jnp.float32,
                            precision=lax.Precision.DEFAULT)
        pn2 = jnp.sum(o * o, axis=0, keepdims=True)
        sq = jnp.maximum(cn2 + pn2 - 2.0 * g, 0.0)
        hinged = jnp.maximum(jnp.sqrt(sq) - _DELTA_A, 0.0)
        hm = jnp.where(t == lab, hinged, 0.0)
        attr = jnp.sum(hm, axis=1, keepdims=True)

        r_i = jnp.float32(0.0)
        for jj in range(k):
            dvec = c2 - lax.slice(c2, (0, jj), (dsz, jj + 1))
            sqd = jnp.sum(dvec * dvec, axis=0, keepdims=True)
            r_i += jnp.sum(jnp.maximum(_DELTA_R - jnp.sqrt(sqd), 0.0)) - _DELTA_R
        g_i = jnp.sum(jnp.sqrt(cn2))
        subl = lax.broadcasted_iota(jnp.int32, (k, 128), 0)
        lanes = lax.broadcasted_iota(jnp.int32, (k, 128), 1)
        vec = (jnp.where(lanes == 0, attr, 0.0)
               + jnp.where((lanes == 1) & (subl == 0), r_i, 0.0)
               + jnp.where((lanes == 2) & (subl == 0), g_i, 0.0))
        res_ref[0] = vec
    return body


def _tc_main(out_r, tgt_r, blk, off):
    b, d, hw = out_r.shape
    k = blk.shape[0] // b

    def mk_in(j):
        return pl.BlockSpec((1, d, 128),
                            lambda i, blk, off, j=j: (i, 0, blk[i * k + j]))

    def mk_tin(j):
        return pl.BlockSpec((1, 1, 128),
                            lambda i, blk, off, j=j: (i, 0, blk[i * k + j]))

    grid_spec = pltpu.PrefetchScalarGridSpec(
        num_scalar_prefetch=2,
        grid=(b,),
        in_specs=([mk_in(j) for j in range(k)]
                  + [mk_tin(j) for j in range(k)]
                  + [pl.BlockSpec((1, d, hw), lambda i, blk, off: (i, 0, 0)),
                     pl.BlockSpec((1, 1, hw), lambda i, blk, off: (i, 0, 0))]),
        out_specs=pl.BlockSpec((1, k, 128), lambda i, blk, off: (i, 0, 0)),
    )
    return pl.pallas_call(
        _make_tc_body(k),
        grid_spec=grid_spec,
        out_shape=jax.ShapeDtypeStruct((b, k, 128), jnp.float32),
        compiler_params=pltpu.CompilerParams(
            dimension_semantics=("arbitrary",),
        ),
    )(blk, off, *([out_r] * k), *([tgt_r] * k), out_r, tgt_r)


def _assemble(res, counts, b, k):
    attr_raw = res[:, :, 0]
    r = res[:, 0, 1]
    g = res[:, 0, 2]
    n = jnp.sum(counts.reshape(b, k, 16), axis=2).astype(jnp.float32)
    denom = jnp.where(n > 1.0, n - 1.0, jnp.maximum(n, 1.0))
    a = jnp.sum(attr_raw / denom, axis=1)
    att = jnp.float32(0.0)
    rep = jnp.float32(0.0)
    reg = jnp.float32(0.0)
    for i in range(b):
        att = (att + a[i]) / k
        rep = (rep + r[i]) / (k * (k - 1))
        reg = (reg + g[i]) / k
    loss = _ALPHA * att + _BETA * rep + _GAMMA * reg
    return (loss, att, rep)


def kernel(out, target, centers, batch_size, device):
    b, d, h, w = out.shape
    k = centers.shape[1]
    hw = h * w

    centers = centers.astype(jnp.int32)
    target = target.astype(jnp.int32)
    p = centers[..., 0] * w + centers[..., 1]
    bidx = jnp.arange(b, dtype=jnp.int32)[:, None]
    lab_idx = (bidx * hw + p).reshape(-1)
    lab_idx_rep = jnp.broadcast_to(lab_idx[:, None], (b * k, 16))

    counts = _sc_counts(target.reshape(-1), lab_idx_rep, b, k, hw)

    p_flat = p.reshape(-1)
    res = _tc_main(out.reshape(b, d, hw), target.reshape(b, 1, hw),
                   p_flat // 128, p_flat % 128)
    return _assemble(res, counts, b, k)
